# baseline (speedup 1.0000x reference)
import jax
import jax.numpy as jnp
from jax import lax
from jax.experimental import pallas as pl
from jax.experimental.pallas import tpu as pltpu
from jax.experimental.pallas import tpu_sc as plsc

_N = 100_000
_E = 6_400_000
_G = 1024
_NC = 2
_NS = 16
_NW = _NC * _NS
_B = 80
_NB = 50
_C = _B * _NB
_NCHUNK = 50
_GB = 1024
_NSLOT = 10


def _sc_body(tab, src2, dst2, out_s, out_c,
             sidx, didx, srows, drows, accs, accc, red, sem_a, sem_b):
    wid = lax.axis_index("s") * _NC + lax.axis_index("c")

    lanes = lax.iota(jnp.int32, 16)
    zeros16 = jnp.zeros((16,), jnp.float32)
    ones16 = jnp.ones((16,), jnp.float32)
    col0 = jnp.zeros((16,), jnp.int32)
    col1 = col0 + 1
    col2 = col0 + 2
    col3 = col0 + 3

    def zero_body(i, _):
        accs[pl.ds(i * 16, 16)] = zeros16
        accc[pl.ds(i * 16, 16)] = zeros16
        return _
    lax.fori_loop(0, 16 * _GB // 16, zero_body, None)

    def compute_slot(b):
        def vec_body(v, _):
            row = b * _B + v * 16 + lanes
            sx = plsc.load_gather(srows, [row, col0])
            sy = plsc.load_gather(srows, [row, col1])
            sz = plsc.load_gather(srows, [row, col2])
            sb = plsc.load_gather(srows, [row, col3])
            tx = plsc.load_gather(drows, [row, col0])
            ty = plsc.load_gather(drows, [row, col1])
            tz = plsc.load_gather(drows, [row, col2])
            dx = sx - tx
            dy = sy - ty
            dz = sz - tz
            n2 = dx * dx + dy * dy + dz * dz
            yi = jnp.int32(0x5F3759DF) - lax.shift_right_arithmetic(
                lax.bitcast_convert_type(n2, jnp.int32), 1)
            y = lax.bitcast_convert_type(yi, jnp.float32)
            y = y * (1.5 - 0.5 * n2 * y * y)
            y = y * (1.5 - 0.5 * n2 * y * y)
            eu = jnp.where(n2 > 0.0, n2 * y, 0.0)
            ev = (eu - 1.0) * (eu - 1.0)
            g = lanes * _GB + sb.astype(jnp.int32)
            plsc.addupdate_scatter(accs, [g], ev)
            plsc.addupdate_scatter(accc, [g], ones16)
            return _
        lax.fori_loop(0, _B // 16, vec_body, None)

    def fire_block(j, b):
        pltpu.async_copy(
            tab.at[sidx.at[j]], srows.at[pl.ds(b * _B, _B)], sem_a.at[b])
        pltpu.async_copy(
            tab.at[didx.at[j]], drows.at[pl.ds(b * _B, _B)], sem_b.at[b])

    def wait_slot(b):
        pltpu.make_async_copy(
            tab.at[pl.ds(0, _B)], srows.at[pl.ds(b * _B, _B)],
            sem_a.at[b]).wait()
        pltpu.make_async_copy(
            tab.at[pl.ds(0, _B)], drows.at[pl.ds(b * _B, _B)],
            sem_b.at[b]).wait()

    def chunk_body(ci, _):
        ro = (wid * _NCHUNK + ci) * _NB
        pltpu.sync_copy(src2.at[pl.ds(ro, _NB)], sidx)
        pltpu.sync_copy(dst2.at[pl.ds(ro, _NB)], didx)
        for b in range(_NSLOT):
            fire_block(b, b)

        def round_body(r, _):
            for b in range(_NSLOT):
                wait_slot(b)
                compute_slot(b)

                @pl.when(r < _NB // _NSLOT - 1)
                def _():
                    fire_block((r + 1) * _NSLOT + b, b)
            return _
        lax.fori_loop(0, _NB // _NSLOT, round_body, None)
        return _
    lax.fori_loop(0, _NCHUNK, chunk_body, None)

    def red_s_body(cb, _):
        c0 = cb * 16
        s = accs[pl.ds(c0, 16)]
        for r in range(1, 16):
            s = s + accs[pl.ds(r * _GB + c0, 16)]
        red[pl.ds(c0, 16)] = s
        return _
    lax.fori_loop(0, _G // 16, red_s_body, None)
    pltpu.sync_copy(red, out_s.at[wid])

    def red_c_body(cb, _):
        c0 = cb * 16
        s = accc[pl.ds(c0, 16)]
        for r in range(1, 16):
            s = s + accc[pl.ds(r * _GB + c0, 16)]
        red[pl.ds(c0, 16)] = s
        return _
    lax.fori_loop(0, _G // 16, red_c_body, None)
    pltpu.sync_copy(red, out_c.at[wid])


_sc_call = pl.kernel(
    _sc_body,
    out_type=(
        jax.ShapeDtypeStruct((_NW, _G), jnp.float32),
        jax.ShapeDtypeStruct((_NW, _G), jnp.float32),
    ),
    mesh=plsc.VectorSubcoreMesh(core_axis_name="c", subcore_axis_name="s"),
    compiler_params=pltpu.CompilerParams(
        needs_layout_passes=False, use_tc_tiling_on_sc=False),
    scratch_types=[
        pltpu.VMEM((_NB, _B), jnp.int32),
        pltpu.VMEM((_NB, _B), jnp.int32),
        pltpu.VMEM((_NSLOT * _B, 8), jnp.float32),
        pltpu.VMEM((_NSLOT * _B, 8), jnp.float32),
        pltpu.VMEM((16 * _GB,), jnp.float32),
        pltpu.VMEM((16 * _GB,), jnp.float32),
        pltpu.VMEM((_G,), jnp.float32),
        pltpu.SemaphoreType.DMA((_NSLOT,)),
        pltpu.SemaphoreType.DMA((_NSLOT,)),
    ],
)


def _final_body(s_ref, c_ref, o_ref):
    sums = jnp.sum(s_ref[...], axis=0)
    cnts = jnp.sum(c_ref[...], axis=0)
    gv = sums / jnp.maximum(cnts, 1.0)
    o_ref[0, 0] = jnp.sum(gv) * (1.0 / _G)


_final_call = pl.pallas_call(
    _final_body,
    out_shape=jax.ShapeDtypeStruct((1, 1), jnp.float32),
    out_specs=pl.BlockSpec(memory_space=pltpu.SMEM),
)


@jax.jit
def kernel(node_pos, raw_edge_index, batch):
    src2 = raw_edge_index[0].astype(jnp.int32).reshape(-1, _B)
    dst2 = raw_edge_index[1].astype(jnp.int32).reshape(-1, _B)
    tab = jnp.concatenate(
        [node_pos.astype(jnp.float32),
         batch.astype(jnp.float32)[:, None],
         jnp.zeros((_N, 4), jnp.float32)], axis=1)
    sums, cnts = _sc_call(tab, src2, dst2)
    out = _final_call(sums, cnts)
    return out[0, 0]


# speedup vs baseline: 1.0097x; 1.0097x over previous
import jax
import jax.numpy as jnp
from jax import lax
from jax.experimental import pallas as pl
from jax.experimental.pallas import tpu as pltpu
from jax.experimental.pallas import tpu_sc as plsc

_N = 100_000
_E = 6_400_000
_G = 1024
_NC = 2
_NS = 16
_NW = _NC * _NS
_B = 80
_NB = 50
_C = _B * _NB
_NCHUNK = 50
_GB = 1024
_NSLOT = 5


def _sc_body(tab, src2, dst2, out_s, out_c,
             sidx, didx, srows, drows, accs, accc, red, sem_a, sem_b):
    wid = lax.axis_index("s") * _NC + lax.axis_index("c")

    lanes = lax.iota(jnp.int32, 16)
    zeros16 = jnp.zeros((16,), jnp.float32)
    ones16 = jnp.ones((16,), jnp.float32)
    col0 = jnp.zeros((16,), jnp.int32)
    col1 = col0 + 1
    col2 = col0 + 2
    col3 = col0 + 3

    def zero_body(i, _):
        accs[pl.ds(i * 16, 16)] = zeros16
        accc[pl.ds(i * 16, 16)] = zeros16
        return _
    lax.fori_loop(0, 16 * _GB // 16, zero_body, None)

    def compute_slot(b):
        def vec_body(v, _):
            row = b * _B + v * 16 + lanes
            sx = plsc.load_gather(srows, [row, col0])
            sy = plsc.load_gather(srows, [row, col1])
            sz = plsc.load_gather(srows, [row, col2])
            sb = plsc.load_gather(srows, [row, col3])
            tx = plsc.load_gather(drows, [row, col0])
            ty = plsc.load_gather(drows, [row, col1])
            tz = plsc.load_gather(drows, [row, col2])
            dx = sx - tx
            dy = sy - ty
            dz = sz - tz
            n2 = dx * dx + dy * dy + dz * dz
            yi = jnp.int32(0x5F3759DF) - lax.shift_right_arithmetic(
                lax.bitcast_convert_type(n2, jnp.int32), 1)
            y = lax.bitcast_convert_type(yi, jnp.float32)
            y = y * (1.5 - 0.5 * n2 * y * y)
            y = y * (1.5 - 0.5 * n2 * y * y)
            y = y * (1.5 - 0.5 * n2 * y * y)
            eu = jnp.where(n2 > 0.0, n2 * y, 0.0)
            ev = (eu - 1.0) * (eu - 1.0)
            g = lanes * _GB + sb.astype(jnp.int32)
            plsc.addupdate_scatter(accs, [g], ev)
            plsc.addupdate_scatter(accc, [g], ones16)
            return _
        lax.fori_loop(0, _B // 16, vec_body, None)

    def fire_block(j, b):
        pltpu.async_copy(
            tab.at[sidx.at[j]], srows.at[pl.ds(b * _B, _B)], sem_a.at[b])
        pltpu.async_copy(
            tab.at[didx.at[j]], drows.at[pl.ds(b * _B, _B)], sem_b.at[b])

    def wait_slot(b):
        pltpu.make_async_copy(
            tab.at[pl.ds(0, _B)], srows.at[pl.ds(b * _B, _B)],
            sem_a.at[b]).wait()
        pltpu.make_async_copy(
            tab.at[pl.ds(0, _B)], drows.at[pl.ds(b * _B, _B)],
            sem_b.at[b]).wait()

    def chunk_body(ci, _):
        ro = (wid * _NCHUNK + ci) * _NB
        pltpu.sync_copy(src2.at[pl.ds(ro, _NB)], sidx)
        pltpu.sync_copy(dst2.at[pl.ds(ro, _NB)], didx)
        for b in range(_NSLOT):
            fire_block(b, b)

        def round_body(r, _):
            for b in range(_NSLOT):
                wait_slot(b)
                compute_slot(b)

                @pl.when(r < _NB // _NSLOT - 1)
                def _():
                    fire_block((r + 1) * _NSLOT + b, b)
            return _
        lax.fori_loop(0, _NB // _NSLOT, round_body, None)
        return _
    lax.fori_loop(0, _NCHUNK, chunk_body, None)

    def red_s_body(cb, _):
        c0 = cb * 16
        s = accs[pl.ds(c0, 16)]
        for r in range(1, 16):
            s = s + accs[pl.ds(r * _GB + c0, 16)]
        red[pl.ds(c0, 16)] = s
        return _
    lax.fori_loop(0, _G // 16, red_s_body, None)
    pltpu.sync_copy(red, out_s.at[wid])

    def red_c_body(cb, _):
        c0 = cb * 16
        s = accc[pl.ds(c0, 16)]
        for r in range(1, 16):
            s = s + accc[pl.ds(r * _GB + c0, 16)]
        red[pl.ds(c0, 16)] = s
        return _
    lax.fori_loop(0, _G // 16, red_c_body, None)
    pltpu.sync_copy(red, out_c.at[wid])


_sc_call = pl.kernel(
    _sc_body,
    out_type=(
        jax.ShapeDtypeStruct((_NW, _G), jnp.float32),
        jax.ShapeDtypeStruct((_NW, _G), jnp.float32),
    ),
    mesh=plsc.VectorSubcoreMesh(core_axis_name="c", subcore_axis_name="s"),
    compiler_params=pltpu.CompilerParams(
        needs_layout_passes=False, use_tc_tiling_on_sc=False),
    scratch_types=[
        pltpu.VMEM((_NB, _B), jnp.int32),
        pltpu.VMEM((_NB, _B), jnp.int32),
        pltpu.VMEM((_NSLOT * _B, 8), jnp.float32),
        pltpu.VMEM((_NSLOT * _B, 8), jnp.float32),
        pltpu.VMEM((16 * _GB,), jnp.float32),
        pltpu.VMEM((16 * _GB,), jnp.float32),
        pltpu.VMEM((_G,), jnp.float32),
        pltpu.SemaphoreType.DMA((_NSLOT,)),
        pltpu.SemaphoreType.DMA((_NSLOT,)),
    ],
)


def _final_body(s_ref, c_ref, o_ref):
    sums = jnp.sum(s_ref[...], axis=0)
    cnts = jnp.sum(c_ref[...], axis=0)
    gv = sums / jnp.maximum(cnts, 1.0)
    o_ref[0, 0] = jnp.sum(gv) * (1.0 / _G)


_final_call = pl.pallas_call(
    _final_body,
    out_shape=jax.ShapeDtypeStruct((1, 1), jnp.float32),
    out_specs=pl.BlockSpec(memory_space=pltpu.SMEM),
)


@jax.jit
def kernel(node_pos, raw_edge_index, batch):
    src2 = raw_edge_index[0].astype(jnp.int32).reshape(-1, _B)
    dst2 = raw_edge_index[1].astype(jnp.int32).reshape(-1, _B)
    tab = jnp.concatenate(
        [node_pos.astype(jnp.float32),
         batch.astype(jnp.float32)[:, None],
         jnp.zeros((_N, 4), jnp.float32)], axis=1)
    sums, cnts = _sc_call(tab, src2, dst2)
    out = _final_call(sums, cnts)
    return out[0, 0]
